# baseline (speedup 1.0000x reference)
import functools

import jax
import jax.numpy as jnp
from jax import lax
from jax.experimental import pallas as pl
from jax.experimental.pallas import tpu as pltpu
from jax.experimental.pallas import tpu_sc as plsc

D = 64
S = 128
LANES = 16
VPR = D // LANES

NC, NS = 2, 16
NW = NC * NS

CHUNK = 512
SEQ_PER_CHUNK = CHUNK // S


def _body(x_hbm, tok_hbm, pos_hbm, out_hbm, idx_v, pos_v, rows_v, sem):
    n_tokens = out_hbm.shape[0]
    per_w = n_tokens // NW
    n_chunks = per_w // CHUNK
    wid = lax.axis_index("s") * NC + lax.axis_index("c")
    base = wid * per_w

    pltpu.sync_copy(pos_hbm, pos_v)

    def chunk_step(g, _):
        off = base + g * CHUNK
        pltpu.sync_copy(x_hbm.at[pl.ds(off, CHUNK)], idx_v)
        pltpu.async_copy(tok_hbm.at[idx_v], rows_v, sem).wait()

        def pos_step(s, _):
            for d in range(VPR):
                pv = pos_v[s, pl.ds(d * LANES, LANES)]
                for q in range(SEQ_PER_CHUNK):
                    plsc.addupdate(rows_v.at[s + q * S, pl.ds(d * LANES, LANES)], pv)
            return ()

        lax.fori_loop(0, S, pos_step, ())
        pltpu.sync_copy(rows_v, out_hbm.at[pl.ds(off, CHUNK)])
        return ()

    lax.fori_loop(0, n_chunks, chunk_step, ())


@jax.jit
def _embed(x_flat, token_table, pos_table):
    n_tokens = x_flat.shape[0]
    kern = pl.kernel(
        _body,
        out_type=jax.ShapeDtypeStruct((n_tokens, D), jnp.float32),
        mesh=plsc.VectorSubcoreMesh(
            core_axis_name="c", subcore_axis_name="s",
            num_cores=NC, num_subcores=NS,
        ),
        scratch_types=[
            pltpu.VMEM((CHUNK,), jnp.int32),
            pltpu.VMEM((S, D), jnp.float32),
            pltpu.VMEM((CHUNK, D), jnp.float32),
            pltpu.SemaphoreType.DMA,
        ],
        compiler_params=pltpu.CompilerParams(use_tc_tiling_on_sc=False),
    )
    return kern(x_flat, token_table, pos_table)


def kernel(x, token_table, pos_table):
    b, s = x.shape
    out = _embed(x.reshape(b * s), token_table, pos_table)
    return out.reshape(b, s, D)


# speedup vs baseline: 1.0801x; 1.0801x over previous
import functools

import jax
import jax.numpy as jnp
from jax import lax
from jax.experimental import pallas as pl
from jax.experimental.pallas import tpu as pltpu
from jax.experimental.pallas import tpu_sc as plsc

D = 64
S = 128
LANES = 16
VPR = D // LANES

NC, NS = 2, 16
NW = NC * NS

CHUNK = 256
SEQ_PER_CHUNK = CHUNK // S
NB = 4


def _body(x_hbm, tok_hbm, pos_hbm, out_hbm, idx_all, pos_v, rows, gsems, ssems):
    n_tokens = out_hbm.shape[0]
    per_w = n_tokens // NW
    n_chunks = per_w // CHUNK
    n_outer = n_chunks // NB
    wid = lax.axis_index("s") * NC + lax.axis_index("c")
    base = wid * per_w

    pltpu.sync_copy(pos_hbm, pos_v)
    pltpu.sync_copy(x_hbm.at[pl.ds(base, per_w)], idx_all)

    def start_gather(b, g):
        pltpu.async_copy(
            tok_hbm.at[idx_all.at[pl.ds(g * CHUNK, CHUNK)]], rows[b], gsems[b])

    def stage(b, g):
        pltpu.make_async_copy(tok_hbm.at[idx_all.at[pl.ds(0, CHUNK)]],
                              rows[b], gsems[b]).wait()

        @pl.loop(0, S, unroll=8)
        def pos_step(s):
            for d in range(VPR):
                pv = pos_v[s, pl.ds(d * LANES, LANES)]
                for q in range(SEQ_PER_CHUNK):
                    plsc.addupdate(
                        rows[b].at[s + q * S, pl.ds(d * LANES, LANES)], pv)

        pltpu.async_copy(
            rows[b], out_hbm.at[pl.ds(base + g * CHUNK, CHUNK)], ssems[b])

    def refill(b, g2):
        pltpu.make_async_copy(
            rows[b], out_hbm.at[pl.ds(base, CHUNK)], ssems[b]).wait()
        start_gather(b, g2)

    for b in range(NB):
        start_gather(b, b)

    def outer(gg, _):
        g0 = gg * NB
        for b in range(NB):
            stage(b, g0 + b)
            if b >= 1:
                last = g0 + (b - 1) + NB

                @pl.when(last < n_chunks)
                def _():
                    refill(b - 1, last)
        last = g0 + (NB - 1) + NB

        @pl.when(last < n_chunks)
        def _():
            refill(NB - 1, last)
        return ()

    lax.fori_loop(0, n_outer, outer, ())
    for b in range(NB):
        pltpu.make_async_copy(
            rows[b], out_hbm.at[pl.ds(base, CHUNK)], ssems[b]).wait()


@jax.jit
def _embed(x_flat, token_table, pos_table):
    n_tokens = x_flat.shape[0]
    per_w = n_tokens // NW
    kern = pl.kernel(
        _body,
        out_type=jax.ShapeDtypeStruct((n_tokens, D), jnp.float32),
        mesh=plsc.VectorSubcoreMesh(
            core_axis_name="c", subcore_axis_name="s",
            num_cores=NC, num_subcores=NS,
        ),
        scratch_types=[
            pltpu.VMEM((per_w,), jnp.int32),
            pltpu.VMEM((S, D), jnp.float32),
            [pltpu.VMEM((CHUNK, D), jnp.float32) for _ in range(NB)],
            [pltpu.SemaphoreType.DMA for _ in range(NB)],
            [pltpu.SemaphoreType.DMA for _ in range(NB)],
        ],
        compiler_params=pltpu.CompilerParams(use_tc_tiling_on_sc=False),
    )
    return kern(x_flat, token_table, pos_table)


def kernel(x, token_table, pos_table):
    b, s = x.shape
    out = _embed(x.reshape(b * s), token_table, pos_table)
    return out.reshape(b, s, D)
